# baseline (speedup 1.0000x reference)
import functools

import jax
import jax.numpy as jnp
from jax import lax
from jax.experimental import pallas as pl
from jax.experimental.pallas import tpu as pltpu
from jax.experimental.pallas import tpu_sc as plsc

_M = 100000
_D = 64
_B = 4096
_NN = 20
_ALPHA = 1.0 / 100.0
_TAU = 200.0

_NCORE = 2
_NSUB = 16
_NW = _NCORE * _NSUB
_JPW = (_B * _NN) // _NW
_SPW = _B // _NW
_GC = 128
_NCH = _JPW // _GC

_mesh = plsc.VectorSubcoreMesh(core_axis_name="c", subcore_axis_name="s",
                               num_cores=_NCORE, num_subcores=_NSUB)

_sc_params = pltpu.CompilerParams(use_tc_tiling_on_sc=False)

_sc_scratch = [
    pltpu.VMEM((_GC,), jnp.int32),
    pltpu.VMEM((_GC,), jnp.int32),
    pltpu.VMEM((_GC, _D), jnp.float32),
    pltpu.VMEM((_GC, _D), jnp.float32),
] + [pltpu.SemaphoreType.DMA] * 6


def _worker_id():
    return lax.axis_index("c") * _NSUB + lax.axis_index("s")


@functools.partial(
    pl.kernel,
    out_type=(
        jax.ShapeDtypeStruct((_B * _NN, _D), jnp.float32),
        jax.ShapeDtypeStruct((_B, _D), jnp.float32),
    ),
    mesh=_mesh,
    scratch_types=_sc_scratch,
    compiler_params=_sc_params,
)
def _sc_gather(mem_hbm, flat_hbm, uid_hbm, nc_hbm, s_hbm,
               i0, i1, r0, r1, si0, si1, sg0, sg1, ss0, ss1):
    wid = _worker_id()
    jb = wid * _JPW
    sb = wid * _SPW
    ibufs, rbufs = (i0, i1), (r0, r1)
    isems, gsems, ssems = (si0, si1), (sg0, sg1), (ss0, ss1)
    cell_hbm = mem_hbm.at[0]

    pltpu.sync_copy(uid_hbm.at[pl.ds(sb, _SPW)], i0)
    pltpu.async_copy(cell_hbm.at[i0], r0, sg0).wait()
    pltpu.sync_copy(r0, s_hbm.at[pl.ds(sb, _SPW)])

    ih = [None] * _NCH
    gh = [None] * _NCH
    sh = [None] * _NCH

    def issue_i(c):
        b = c % 2
        ih[c] = pltpu.async_copy(
            flat_hbm.at[pl.ds(jb + c * _GC, _GC)], ibufs[b], isems[b])

    def issue_g(c):
        b = c % 2
        gh[c] = pltpu.async_copy(cell_hbm.at[ibufs[b]], rbufs[b], gsems[b])

    def issue_s(c):
        b = c % 2
        sh[c] = pltpu.async_copy(
            rbufs[b], nc_hbm.at[pl.ds(jb + c * _GC, _GC)], ssems[b])

    issue_i(0)
    issue_i(1)
    ih[0].wait()
    issue_g(0)
    ih[1].wait()
    issue_g(1)
    for c in range(_NCH):
        gh[c].wait()
        issue_s(c)
        sh[c].wait()
        if c + 2 < _NCH:
            issue_i(c + 2)
            ih[c + 2].wait()
            issue_g(c + 2)


@functools.partial(
    pl.kernel,
    out_type=(),
    mesh=_mesh,
    scratch_types=_sc_scratch,
    compiler_params=_sc_params,
)
def _sc_scatter(o_hbm, cv_hbm, hv_hbm, f_hbm,
                i0, i1, r0, r1, si0, si1, sr0, sr1, sw0, sw1):
    wid = _worker_id()
    jb = wid * _JPW
    ibufs, rbufs = (i0, i1), (r0, r1)
    isems, rsems, wsems = (si0, si1), (sr0, sr1), (sw0, sw1)

    for src_hbm, plane in ((cv_hbm, 0), (hv_hbm, 1)):
        dst_hbm = o_hbm.at[plane]
        ih = [None] * _NCH
        rh = [None] * _NCH
        wh = [None] * _NCH

        def issue_load(c, src_hbm=src_hbm, ih=ih, rh=rh):
            b = c % 2
            ih[c] = pltpu.async_copy(
                f_hbm.at[pl.ds(jb + c * _GC, _GC)], ibufs[b], isems[b])
            rh[c] = pltpu.async_copy(
                src_hbm.at[pl.ds(jb + c * _GC, _GC)], rbufs[b], rsems[b])

        issue_load(0)
        issue_load(1)
        for c in range(_NCH):
            b = c % 2
            ih[c].wait()
            rh[c].wait()
            wh[c] = pltpu.async_copy(rbufs[b], dst_hbm.at[ibufs[b]], wsems[b])
            wh[c].wait()
            if c + 2 < _NCH:
                issue_load(c + 2)


def _proj_body(m_ref, w_ref, o_ref):
    o_ref[...] = jnp.dot(m_ref[...], w_ref[...],
                         preferred_element_type=jnp.float32)


def _tc_project(msgs, w):
    return pl.pallas_call(
        _proj_body,
        out_shape=jax.ShapeDtypeStruct((5 * _B, _D), jnp.float32),
        grid=(5,),
        in_specs=[
            pl.BlockSpec((_B, _D), lambda i: (0, 0)),
            pl.BlockSpec((_D, _D), lambda i: (0, 0)),
        ],
        out_specs=pl.BlockSpec((_B, _D), lambda i: (i, 0)),
    )(msgs, w)


_BB = 256
_BR = _BB * _NN


def _unpack128(x, n):
    l = x[:, :_D]
    r = x[:, _D:]
    return jnp.concatenate([l[:, None, :], r[:, None, :]], axis=1)


def _pack128(x3):
    return jnp.concatenate([x3[:, 0, :], x3[:, 1, :]], axis=1)


def _compute_body(nc_ref, s_ref, ts_ref, ets_ref, p_ref, cv_ref, hv_ref):
    nc3 = _unpack128(nc_ref[...], _BR).reshape(_BB, _NN, _D)
    s = _unpack128(s_ref[...], _BB).reshape(_BB, _D)
    logits = jnp.sum(nc3 * s[:, None, :], axis=2)
    m = jnp.max(logits, axis=1, keepdims=True)
    e = jnp.exp(logits - m)
    att = e / jnp.sum(e, axis=1, keepdims=True)
    delta = ts_ref[...] - ets_ref[...]
    ok = jnp.logical_and(delta > 0.0, delta < _TAU)
    coef = jnp.where(ok, jnp.exp(-_ALPHA * delta) * att, 0.0)
    p3 = p_ref[...].reshape(_BB, _NN, _D)
    cv3 = nc3 + coef[:, :, None] * p3
    cv_ref[...] = _pack128(cv3.reshape(_BR // 2, 2, _D))
    hv_ref[...] = _pack128(jnp.tanh(cv3).reshape(_BR // 2, 2, _D))


def _tc_compute(nc128, s128, ts2, ets, p5):
    grid = _B // _BB
    sds = jax.ShapeDtypeStruct((_B * _NN // 2, 2 * _D), jnp.float32)
    return pl.pallas_call(
        _compute_body,
        out_shape=(sds, sds),
        grid=(grid,),
        in_specs=[
            pl.BlockSpec((_BR // 2, 2 * _D), lambda k: (k, 0)),
            pl.BlockSpec((_BB // 2, 2 * _D), lambda k: (k, 0)),
            pl.BlockSpec((_BB, 1), lambda k: (k, 0)),
            pl.BlockSpec((_BB, _NN), lambda k: (k, 0)),
            pl.BlockSpec((_BR, _D), lambda k: (lax.rem(k, 4), 0)),
        ],
        out_specs=(
            pl.BlockSpec((_BR // 2, 2 * _D), lambda k: (k, 0)),
            pl.BlockSpec((_BR // 2, 2 * _D), lambda k: (k, 0)),
        ),
    )(nc128, s128, ts2, ets, p5)


def kernel(memory, unique_node_ids, unique_messages, timestamps,
           neighbors, edge_times, W_s):
    flat = neighbors.reshape(-1).astype(jnp.int32)
    uids = unique_node_ids.astype(jnp.int32)

    p5 = _tc_project(unique_messages, W_s)
    nc_flat, s_flat = _sc_gather(memory, flat, uids)
    ts2 = timestamps.reshape(_B, 1)

    nc128 = nc_flat.reshape(_B * _NN // 2, 2 * _D)
    s128 = s_flat.reshape(_B // 2, 2 * _D)

    cv128, hv128 = _tc_compute(nc128, s128, ts2, edge_times, p5)
    cv = cv128.reshape(_B * _NN, _D)
    hv = hv128.reshape(_B * _NN, _D)

    out_ref = jax.new_ref(memory)
    _sc_scatter(out_ref, cv, hv, flat)
    return jax.freeze(out_ref)


# speedup vs baseline: 1.1307x; 1.1307x over previous
import functools

import jax
import jax.numpy as jnp
from jax import lax
from jax.experimental import pallas as pl
from jax.experimental.pallas import tpu as pltpu
from jax.experimental.pallas import tpu_sc as plsc

_M = 100000
_D = 64
_B = 4096
_NN = 20
_ALPHA = 1.0 / 100.0
_TAU = 200.0

_NCORE = 2
_NSUB = 16
_NW = _NCORE * _NSUB
_JPW = (_B * _NN) // _NW
_SPW = (2 * _B) // _NW
_GC = 128
_NCH = _JPW // _GC
_SCH = _SPW // _GC

_mesh = plsc.VectorSubcoreMesh(core_axis_name="c", subcore_axis_name="s",
                               num_cores=_NCORE, num_subcores=_NSUB)

_sc_params = pltpu.CompilerParams(use_tc_tiling_on_sc=False)

_sc_scratch = [
    pltpu.VMEM((_GC,), jnp.int32),
    pltpu.VMEM((_GC,), jnp.int32),
    pltpu.VMEM((_GC, _D), jnp.float32),
    pltpu.VMEM((_GC, _D), jnp.float32),
] + [pltpu.SemaphoreType.DMA] * 6


def _worker_id():
    return lax.axis_index("c") * _NSUB + lax.axis_index("s")


@functools.partial(
    pl.kernel,
    out_type=(
        jax.ShapeDtypeStruct((_B * _NN, _D), jnp.float32),
        jax.ShapeDtypeStruct((2 * _B, _D), jnp.float32),
    ),
    mesh=_mesh,
    scratch_types=_sc_scratch,
    compiler_params=_sc_params,
)
def _sc_gather(mem_hbm, flat_hbm, uid2_hbm, nc_hbm, s_hbm,
               i0, i1, r0, r1, si0, si1, sg0, sg1, ss0, ss1):
    wid = _worker_id()
    jb = wid * _JPW
    sb = wid * _SPW
    ibufs, rbufs = (i0, i1), (r0, r1)
    isems, gsems, ssems = (si0, si1), (sg0, sg1), (ss0, ss1)
    cell_hbm = mem_hbm.at[0]

    for c in range(_SCH):
        pltpu.sync_copy(uid2_hbm.at[pl.ds(sb + c * _GC, _GC)], i0)
        pltpu.async_copy(cell_hbm.at[i0], r0, sg0).wait()
        pltpu.sync_copy(r0, s_hbm.at[pl.ds(sb + c * _GC, _GC)])

    ih = [None] * _NCH
    gh = [None] * _NCH
    sh = [None] * _NCH

    def issue_i(c):
        b = c % 2
        ih[c] = pltpu.async_copy(
            flat_hbm.at[pl.ds(jb + c * _GC, _GC)], ibufs[b], isems[b])

    def issue_g(c):
        b = c % 2
        gh[c] = pltpu.async_copy(cell_hbm.at[ibufs[b]], rbufs[b], gsems[b])

    def issue_s(c):
        b = c % 2
        sh[c] = pltpu.async_copy(
            rbufs[b], nc_hbm.at[pl.ds(jb + c * _GC, _GC)], ssems[b])

    issue_i(0)
    issue_i(1)
    ih[0].wait()
    issue_g(0)
    ih[1].wait()
    issue_g(1)
    for c in range(_NCH):
        gh[c].wait()
        issue_s(c)
        sh[c].wait()
        if c + 2 < _NCH:
            issue_i(c + 2)
            ih[c + 2].wait()
            issue_g(c + 2)


@functools.partial(
    pl.kernel,
    out_type=(),
    mesh=_mesh,
    scratch_types=_sc_scratch,
    compiler_params=_sc_params,
)
def _sc_scatter(o_hbm, cv_hbm, hv_hbm, f_hbm,
                i0, i1, r0, r1, si0, si1, sr0, sr1, sw0, sw1):
    wid = _worker_id()
    jb = wid * _JPW
    ibufs, rbufs = (i0, i1), (r0, r1)
    isems, rsems, wsems = (si0, si1), (sr0, sr1), (sw0, sw1)

    for src_hbm, plane in ((cv_hbm, 0), (hv_hbm, 1)):
        dst_hbm = o_hbm.at[plane]
        ih = [None] * _NCH
        rh = [None] * _NCH
        wh = [None] * _NCH

        def issue_load(c, src_hbm=src_hbm, ih=ih, rh=rh):
            b = c % 2
            ih[c] = pltpu.async_copy(
                f_hbm.at[pl.ds(jb + c * _GC, _GC)], ibufs[b], isems[b])
            rh[c] = pltpu.async_copy(
                src_hbm.at[pl.ds(jb + c * _GC, _GC)], rbufs[b], rsems[b])

        issue_load(0)
        issue_load(1)
        for c in range(_NCH):
            b = c % 2
            ih[c].wait()
            rh[c].wait()
            wh[c] = pltpu.async_copy(rbufs[b], dst_hbm.at[ibufs[b]], wsems[b])
            wh[c].wait()
            if c + 2 < _NCH:
                issue_load(c + 2)


def _proj_body(m_ref, w_ref, o_ref):
    o_ref[...] = jnp.dot(m_ref[...], w_ref[...],
                         preferred_element_type=jnp.float32)


def _tc_project(msgs2, w2):
    return pl.pallas_call(
        _proj_body,
        out_shape=jax.ShapeDtypeStruct((5 * _B // 2, 2 * _D), jnp.float32),
        grid=(5,),
        in_specs=[
            pl.BlockSpec((_B // 2, 2 * _D), lambda i: (0, 0)),
            pl.BlockSpec((2 * _D, 2 * _D), lambda i: (0, 0)),
        ],
        out_specs=pl.BlockSpec((_B // 2, 2 * _D), lambda i: (i, 0)),
    )(msgs2, w2)


_BB = 256
_BP = _BB * _NN // 2
_NP = _NN // 2


def _compute_body(nc_ref, s_ref, ts_ref, ets_ref, p_ref, cv_ref, hv_ref):
    nc = nc_ref[...]
    sb = s_ref[...]
    s_pair = jnp.broadcast_to(
        sb[:, None, :], (_BB, _NP, 2 * _D)).reshape(_BP, 2 * _D)
    z = nc * s_pair
    l = jnp.sum(z[:, :_D], axis=1, keepdims=True)
    r = jnp.sum(z[:, _D:], axis=1, keepdims=True)
    m = jnp.max(jnp.maximum(l, r))
    el = jnp.exp(l - m)
    er = jnp.exp(r - m)
    st = el + er
    bi = lax.broadcasted_iota(jnp.int32, (_BB, _BP), 0)
    ti = lax.broadcasted_iota(jnp.int32, (_BB, _BP), 1)
    seg = jnp.where(ti // _NP == bi, 1.0, 0.0)
    dens = jnp.dot(seg, st, preferred_element_type=jnp.float32)
    inv = 1.0 / dens
    invp = jnp.broadcast_to(inv[:, None, :], (_BB, _NP, 1)).reshape(_BP, 1)
    tsp = jnp.broadcast_to(
        ts_ref[...][:, None, :], (_BB, _NP, 1)).reshape(_BP, 1)
    delta = tsp - ets_ref[...]
    ok = jnp.logical_and(delta > 0.0, delta < _TAU)
    att2 = jnp.concatenate([el, er], axis=1) * invp
    coef2 = jnp.where(ok, jnp.exp(-_ALPHA * delta) * att2, 0.0)
    c128 = jnp.concatenate([
        jnp.broadcast_to(coef2[:, 0:1], (_BP, _D)),
        jnp.broadcast_to(coef2[:, 1:2], (_BP, _D)),
    ], axis=1)
    cv = nc + c128 * p_ref[...]
    cv_ref[...] = cv
    hv_ref[...] = jnp.tanh(cv)


def _tc_compute(nc128, s128, ts2, ets2, p5):
    grid = _B // _BB
    sds = jax.ShapeDtypeStruct((_B * _NN // 2, 2 * _D), jnp.float32)
    return pl.pallas_call(
        _compute_body,
        out_shape=(sds, sds),
        grid=(grid,),
        in_specs=[
            pl.BlockSpec((_BP, 2 * _D), lambda k: (k, 0)),
            pl.BlockSpec((_BB, 2 * _D), lambda k: (k, 0)),
            pl.BlockSpec((_BB, 1), lambda k: (k, 0)),
            pl.BlockSpec((_BP, 2), lambda k: (k, 0)),
            pl.BlockSpec((_BP, 2 * _D), lambda k: (lax.rem(k, 4), 0)),
        ],
        out_specs=(
            pl.BlockSpec((_BP, 2 * _D), lambda k: (k, 0)),
            pl.BlockSpec((_BP, 2 * _D), lambda k: (k, 0)),
        ),
    )(nc128, s128, ts2, ets2, p5)


def kernel(memory, unique_node_ids, unique_messages, timestamps,
           neighbors, edge_times, W_s):
    flat = neighbors.reshape(-1).astype(jnp.int32)
    uids = unique_node_ids.astype(jnp.int32)
    uid2 = jnp.repeat(uids, 2)

    msgs2 = unique_messages.reshape(_B // 2, 2 * _D)
    w2 = jnp.zeros((2 * _D, 2 * _D), jnp.float32)
    w2 = w2.at[:_D, :_D].set(W_s).at[_D:, _D:].set(W_s)
    p5 = _tc_project(msgs2, w2)

    nc_flat, s_flat = _sc_gather(memory, flat, uid2)

    nc128 = nc_flat.reshape(_B * _NN // 2, 2 * _D)
    s128 = s_flat.reshape(_B, 2 * _D)
    ts2 = timestamps.reshape(_B, 1)
    ets2 = edge_times.reshape(_B * _NN // 2, 2)

    cv128, hv128 = _tc_compute(nc128, s128, ts2, ets2, p5)
    cv = cv128.reshape(_B * _NN, _D)
    hv = hv128.reshape(_B * _NN, _D)

    out_ref = jax.new_ref(memory)
    _sc_scatter(out_ref, cv, hv, flat)
    return jax.freeze(out_ref)


# speedup vs baseline: 1.3187x; 1.1662x over previous
import functools

import jax
import jax.numpy as jnp
from jax import lax
from jax.experimental import pallas as pl
from jax.experimental.pallas import tpu as pltpu
from jax.experimental.pallas import tpu_sc as plsc

_M = 100000
_D = 64
_B = 4096
_NN = 20
_ALPHA = 1.0 / 100.0
_TAU = 200.0

_NCORE = 2
_NSUB = 16
_NW = _NCORE * _NSUB
_JPW = (_B * _NN) // _NW
_SPW = (2 * _B) // _NW
_GC = 128
_NCH = _JPW // _GC
_SCH = _SPW // _GC

_mesh = plsc.VectorSubcoreMesh(core_axis_name="c", subcore_axis_name="s",
                               num_cores=_NCORE, num_subcores=_NSUB)

_sc_params = pltpu.CompilerParams(use_tc_tiling_on_sc=False)

_sc_scratch = [
    pltpu.VMEM((_GC,), jnp.int32),
    pltpu.VMEM((_GC,), jnp.int32),
    pltpu.VMEM((_GC, _D), jnp.float32),
    pltpu.VMEM((_GC, _D), jnp.float32),
] + [pltpu.SemaphoreType.DMA] * 6


def _worker_id():
    return lax.axis_index("c") * _NSUB + lax.axis_index("s")


@functools.partial(
    pl.kernel,
    out_type=(
        jax.ShapeDtypeStruct((_B * _NN, _D), jnp.float32),
        jax.ShapeDtypeStruct((2 * _B, _D), jnp.float32),
    ),
    mesh=_mesh,
    scratch_types=_sc_scratch,
    compiler_params=_sc_params,
)
def _sc_gather(mem_hbm, flat_hbm, uid2_hbm, nc_hbm, s_hbm,
               i0, i1, r0, r1, si0, si1, sg0, sg1, ss0, ss1):
    wid = _worker_id()
    jb = wid * _JPW
    sb = wid * _SPW
    ibufs, rbufs = (i0, i1), (r0, r1)
    isems, gsems, ssems = (si0, si1), (sg0, sg1), (ss0, ss1)
    cell_hbm = mem_hbm.at[0]

    for c in range(_SCH):
        pltpu.sync_copy(uid2_hbm.at[pl.ds(sb + c * _GC, _GC)], i0)
        pltpu.async_copy(cell_hbm.at[i0], r0, sg0).wait()
        pltpu.sync_copy(r0, s_hbm.at[pl.ds(sb + c * _GC, _GC)])

    ih = [None] * _NCH
    gh = [None] * _NCH
    sh = [None] * _NCH

    def issue_i(c):
        b = c % 2
        ih[c] = pltpu.async_copy(
            flat_hbm.at[pl.ds(jb + c * _GC, _GC)], ibufs[b], isems[b])

    def issue_g(c):
        b = c % 2
        gh[c] = pltpu.async_copy(cell_hbm.at[ibufs[b]], rbufs[b], gsems[b])

    def issue_s(c):
        b = c % 2
        sh[c] = pltpu.async_copy(
            rbufs[b], nc_hbm.at[pl.ds(jb + c * _GC, _GC)], ssems[b])

    issue_i(0)
    issue_i(1)
    ih[0].wait()
    issue_g(0)
    ih[1].wait()
    issue_g(1)
    for c in range(_NCH):
        gh[c].wait()
        issue_s(c)
        sh[c].wait()
        if c + 2 < _NCH:
            issue_i(c + 2)
            ih[c + 2].wait()
            issue_g(c + 2)


@functools.partial(
    pl.kernel,
    out_type=(),
    mesh=_mesh,
    scratch_types=_sc_scratch,
    compiler_params=_sc_params,
)
def _sc_scatter(o_hbm, cv_hbm, hv_hbm, f_hbm,
                i0, i1, r0, r1, si0, si1, sr0, sr1, sw0, sw1):
    wid = _worker_id()
    jb = wid * _JPW
    ibufs, rbufs = (i0, i1), (r0, r1)
    isems, rsems, wsems = (si0, si1), (sr0, sr1), (sw0, sw1)

    for src_hbm, plane in ((cv_hbm, 0), (hv_hbm, 1)):
        dst_hbm = o_hbm.at[plane]
        ih = [None] * _NCH
        rh = [None] * _NCH
        wh = [None] * _NCH

        def issue_load(c, src_hbm=src_hbm, ih=ih, rh=rh):
            b = c % 2
            ih[c] = pltpu.async_copy(
                f_hbm.at[pl.ds(jb + c * _GC, _GC)], ibufs[b], isems[b])
            rh[c] = pltpu.async_copy(
                src_hbm.at[pl.ds(jb + c * _GC, _GC)], rbufs[b], rsems[b])

        issue_load(0)
        issue_load(1)
        for c in range(_NCH):
            b = c % 2
            ih[c].wait()
            rh[c].wait()
            wh[c] = pltpu.async_copy(rbufs[b], dst_hbm.at[ibufs[b]], wsems[b])
            wh[c].wait()
            if c + 2 < _NCH:
                issue_load(c + 2)


def _proj_body(m_ref, w_ref, o_ref):
    o_ref[...] = jnp.dot(m_ref[...], w_ref[...],
                         preferred_element_type=jnp.float32)


def _tc_project(msgs2, w2):
    return pl.pallas_call(
        _proj_body,
        out_shape=jax.ShapeDtypeStruct((5 * _B // 2, 2 * _D), jnp.float32),
        grid=(5,),
        in_specs=[
            pl.BlockSpec((_B // 2, 2 * _D), lambda i: (0, 0)),
            pl.BlockSpec((2 * _D, 2 * _D), lambda i: (0, 0)),
        ],
        out_specs=pl.BlockSpec((_B // 2, 2 * _D), lambda i: (i, 0)),
    )(msgs2, w2)


_BB = 256
_BP = _BB * _NN // 2
_NP = _NN // 2


def _dotf(a, b):
    return jnp.dot(a, b, preferred_element_type=jnp.float32)


def _compute_body(nc_ref, s_ref, ts_ref, ets_ref, p_ref, cv_ref, hv_ref):
    f32 = jnp.float32
    nc = nc_ref[...]
    sb = s_ref[...]
    s_pair = jnp.broadcast_to(
        sb[:, None, :], (_BB, _NP, 2 * _D)).reshape(_BP, 2 * _D)

    ri = lax.broadcasted_iota(jnp.int32, (2 * _D, 2 * _D), 0)
    ci = lax.broadcasted_iota(jnp.int32, (2 * _D, 2 * _D), 1)
    blkdiag = jnp.where((ri < _D) == (ci < _D), 1.0, 0.0)
    swap = jnp.where(jnp.abs(ri - ci) == _D, 1.0, 0.0)
    bi = lax.broadcasted_iota(jnp.int32, (_BB, _BP), 0)
    ti = lax.broadcasted_iota(jnp.int32, (_BB, _BP), 1)
    seg = jnp.where(ti // _NP == bi, 1.0, 0.0)
    ti2 = lax.broadcasted_iota(jnp.int32, (_BP, _BB), 0)
    bi2 = lax.broadcasted_iota(jnp.int32, (_BP, _BB), 1)
    segT = jnp.where(ti2 // _NP == bi2, 1.0, 0.0)
    hi = lax.broadcasted_iota(jnp.int32, (2, 2 * _D), 0)
    cj = lax.broadcasted_iota(jnp.int32, (2, 2 * _D), 1)
    widen2 = jnp.where(cj // _D == hi, 1.0, 0.0)

    z = nc * s_pair
    lw = _dotf(z, blkdiag)
    m = jnp.max(lw)
    ew = jnp.exp(lw - m)
    densw = _dotf(seg, ew)
    denst = densw + _dotf(densw, swap)
    invw = 1.0 / denst
    invp = _dotf(segT, invw)
    tsp = _dotf(segT, ts_ref[...])
    etsw = _dotf(ets_ref[...], widen2)
    delta = tsp - etsw
    ok = jnp.logical_and(delta > 0.0, delta < _TAU)
    coefw = jnp.where(ok, jnp.exp(-_ALPHA * delta) * ew * invp, 0.0)
    cv = nc + coefw * p_ref[...]
    cv_ref[...] = cv
    hv_ref[...] = jnp.tanh(cv)


def _tc_compute(nc128, s128, ts128, ets2, p5):
    grid = _B // _BB
    sds = jax.ShapeDtypeStruct((_B * _NN // 2, 2 * _D), jnp.float32)
    return pl.pallas_call(
        _compute_body,
        out_shape=(sds, sds),
        grid=(grid,),
        in_specs=[
            pl.BlockSpec((_BP, 2 * _D), lambda k: (k, 0)),
            pl.BlockSpec((_BB, 2 * _D), lambda k: (k, 0)),
            pl.BlockSpec((_BB, 2 * _D), lambda k: (k, 0)),
            pl.BlockSpec((_BP, 2), lambda k: (k, 0)),
            pl.BlockSpec((_BP, 2 * _D), lambda k: (lax.rem(k, 4), 0)),
        ],
        out_specs=(
            pl.BlockSpec((_BP, 2 * _D), lambda k: (k, 0)),
            pl.BlockSpec((_BP, 2 * _D), lambda k: (k, 0)),
        ),
    )(nc128, s128, ts128, ets2, p5)


def kernel(memory, unique_node_ids, unique_messages, timestamps,
           neighbors, edge_times, W_s):
    flat = neighbors.reshape(-1).astype(jnp.int32)
    uids = unique_node_ids.astype(jnp.int32)
    uid2 = jnp.repeat(uids, 2)

    msgs2 = unique_messages.reshape(_B // 2, 2 * _D)
    w2 = jnp.zeros((2 * _D, 2 * _D), jnp.float32)
    w2 = w2.at[:_D, :_D].set(W_s).at[_D:, _D:].set(W_s)
    p5 = _tc_project(msgs2, w2)

    nc_flat, s_flat = _sc_gather(memory, flat, uid2)

    nc128 = nc_flat.reshape(_B * _NN // 2, 2 * _D)
    s128 = s_flat.reshape(_B, 2 * _D)
    ts128 = jnp.broadcast_to(timestamps[:, None], (_B, 2 * _D))
    ets2 = edge_times.reshape(_B * _NN // 2, 2)

    cv128, hv128 = _tc_compute(nc128, s128, ts128, ets2, p5)
    cv = cv128.reshape(_B * _NN, _D)
    hv = hv128.reshape(_B * _NN, _D)

    out_ref = jax.new_ref(memory)
    _sc_scatter(out_ref, cv, hv, flat)
    return jax.freeze(out_ref)
